# baseline (speedup 1.0000x reference)
import functools

import numpy as np
import jax
import jax.numpy as jnp
from jax import lax
from jax.experimental import pallas as pl
from jax.experimental.pallas import tpu as pltpu
from jax.experimental.pallas import tpu_sc as plsc

_L = 16
_BIG = 1 << 22


def _places_body(hc_ref, vr_ref, places_ref):
    places_ref[0] = hc_ref[0] * vr_ref[0]


def _make_sc_sampled(BP, K, maps_per_w):
    mapsz = K * K
    nchunks = K // _L

    def body(packed_ref, zeros_ref, out_ref, rowbuf, zbuf, idxbuf, valbuf,
             zsem, ssem):
        wid = lax.axis_index("c") * 16 + lax.axis_index("s")
        pltpu.sync_copy(zeros_ref, zbuf)

        lane = jax.lax.iota(jnp.int32, _L)

        def winners(off):
            mvec = jnp.full((_L,), -jnp.inf, jnp.float32)
            for c in range(nchunks):
                mvec = jnp.maximum(mvec, rowbuf[pl.ds(off + c * _L, _L)])
            m = jnp.max(mvec)
            c1 = jnp.full((_L,), _BIG, jnp.int32)
            for c in range(nchunks):
                x = rowbuf[pl.ds(off + c * _L, _L)]
                idx = lane + (c * _L)
                c1 = jnp.minimum(c1, jnp.where(x == m, idx, _BIG))
            i1 = jnp.min(c1)
            c2 = jnp.full((_L,), _BIG, jnp.int32)
            for c in range(nchunks):
                x = rowbuf[pl.ds(off + c * _L, _L)]
                idx = lane + (c * _L)
                c2 = jnp.minimum(
                    c2, jnp.where((x == m) & (idx != i1), idx, _BIG))
            i2 = jnp.min(c2)
            return i1, i2

        def per_map(mi, carry):
            m = wid * maps_per_w + mi
            zcopy = pltpu.make_async_copy(
                zbuf, out_ref.at[pl.ds(m * mapsz, mapsz)], zsem)
            zcopy.start()
            pltpu.sync_copy(packed_ref.at[pl.ds(m * (4 * K), 4 * K)], rowbuf)
            i1, i2 = winners(2 * K)
            j1, j2 = winners(3 * K)
            i2e = jnp.where(i2 < _BIG, i2, i1)
            j2e = jnp.where(j2 < _BIG, j2, j1)
            ri = jnp.where(((lane >> 1) & 1) == 1, i2e, i1)
            cj = jnp.where((lane & 1) == 1, j2e, j1)
            hval = plsc.load_gather(rowbuf, [ri])
            vval = plsc.load_gather(rowbuf, [cj + K])
            idxbuf[...] = (m * mapsz) + ri * K + cj
            valbuf[...] = (hval * vval) * 100.0
            zcopy.wait()
            pltpu.async_copy(valbuf, out_ref.at[idxbuf], ssem).wait()
            return carry

        lax.fori_loop(0, maps_per_w, per_map, 0)

    mesh = plsc.VectorSubcoreMesh(core_axis_name="c", subcore_axis_name="s")
    return functools.partial(
        pl.kernel, body, mesh=mesh,
        compiler_params=pltpu.CompilerParams(needs_layout_passes=False),
        out_type=jax.ShapeDtypeStruct((BP * mapsz,), jnp.float32),
        scratch_types=[
            pltpu.VMEM((4 * K,), jnp.float32),
            pltpu.VMEM((mapsz,), jnp.float32),
            pltpu.VMEM((_L,), jnp.int32),
            pltpu.VMEM((_L,), jnp.float32),
            pltpu.SemaphoreType.DMA,
            pltpu.SemaphoreType.DMA,
        ])()


def kernel(x_cat):
    B, A, K = x_cat.shape
    P = A // 2
    BP = B * P
    f32 = jnp.float32

    t = 0
    beta = 0.1 + 0.1 * np.sin(t / 1000)
    nkey = jax.random.key(42)
    noise = jnp.stack(
        [beta * jax.random.gumbel(jax.random.fold_in(nkey, j), (B, K), dtype=f32)
         for j in range(A)], axis=1)

    h = x_cat[:, 0::2, :]
    v = x_cat[:, 1::2, :]
    log_h = jnp.log(h) + noise[:, 0::2, :]
    log_v = jnp.log(v) + noise[:, 1::2, :]

    hc = h[..., None]
    vr = v[:, :, None, :]
    places = pl.pallas_call(
        _places_body,
        grid=(B,),
        in_specs=[pl.BlockSpec((1, P, K, 1), lambda b: (b, 0, 0, 0)),
                  pl.BlockSpec((1, P, 1, K), lambda b: (b, 0, 0, 0))],
        out_specs=pl.BlockSpec((1, P, K, K), lambda b: (b, 0, 0, 0)),
        out_shape=jax.ShapeDtypeStruct((B, P, K, K), f32),
        compiler_params=pltpu.CompilerParams(
            dimension_semantics=("parallel",)),
    )(hc, vr)

    packed = jnp.stack(
        [h.reshape(BP, K), v.reshape(BP, K),
         log_h.reshape(BP, K), log_v.reshape(BP, K)], axis=1).reshape(-1)
    zeros = jnp.zeros((K * K,), f32)
    sampled = _make_sc_sampled(BP, K, BP // 32)(packed, zeros)
    return (places, sampled.reshape(B, P, K, K))


# speedup vs baseline: 1.0014x; 1.0014x over previous
import functools

import numpy as np
import jax
import jax.numpy as jnp
from jax import lax
from jax.experimental import pallas as pl
from jax.experimental.pallas import tpu as pltpu
from jax.experimental.pallas import tpu_sc as plsc

_L = 16
_BIG = 1 << 22


def _places_body(hc_ref, vr_ref, places_ref):
    places_ref[0] = hc_ref[0] * vr_ref[0]


def _make_sc_sampled(BP, K, maps_per_w):
    mapsz = K * K
    nchunks = K // _L

    def body(packed_ref, zeros_ref, out_ref, rowbuf, zbuf, idxbuf, valbuf,
             zsem, ssem):
        wid = lax.axis_index("c") * 16 + lax.axis_index("s")
        base_map = wid * maps_per_w
        pltpu.sync_copy(zeros_ref, zbuf)
        pltpu.sync_copy(
            packed_ref.at[pl.ds(base_map * (4 * K), maps_per_w * 4 * K)],
            rowbuf)
        zcopies = []
        for mi in range(maps_per_w):
            c = pltpu.make_async_copy(
                zbuf, out_ref.at[pl.ds((base_map + mi) * mapsz, mapsz)], zsem)
            c.start()
            zcopies.append(c)

        lane = jax.lax.iota(jnp.int32, _L)

        def winners(off):
            mvec = jnp.full((_L,), -jnp.inf, jnp.float32)
            for c in range(nchunks):
                mvec = jnp.maximum(mvec, rowbuf[pl.ds(off + c * _L, _L)])
            m = jnp.max(mvec)
            c1 = jnp.full((_L,), _BIG, jnp.int32)
            for c in range(nchunks):
                x = rowbuf[pl.ds(off + c * _L, _L)]
                idx = lane + (c * _L)
                c1 = jnp.minimum(c1, jnp.where(x == m, idx, _BIG))
            i1 = jnp.min(c1)
            c2 = jnp.full((_L,), _BIG, jnp.int32)
            for c in range(nchunks):
                x = rowbuf[pl.ds(off + c * _L, _L)]
                idx = lane + (c * _L)
                c2 = jnp.minimum(
                    c2, jnp.where((x == m) & (idx != i1), idx, _BIG))
            i2 = jnp.min(c2)
            return i1, i2

        scopies = []
        for mi in range(maps_per_w):
            roff = mi * 4 * K
            i1, i2 = winners(roff + 2 * K)
            j1, j2 = winners(roff + 3 * K)
            i2e = jnp.where(i2 < _BIG, i2, i1)
            j2e = jnp.where(j2 < _BIG, j2, j1)
            ri = jnp.where(((lane >> 1) & 1) == 1, i2e, i1)
            cj = jnp.where((lane & 1) == 1, j2e, j1)
            hval = plsc.load_gather(rowbuf, [ri + roff])
            vval = plsc.load_gather(rowbuf, [cj + (roff + K)])
            slot = mi % 2
            if len(scopies) >= 2:
                scopies[-2].wait()
            idxbuf[slot, :] = ((base_map + mi) * mapsz) + ri * K + cj
            valbuf[slot, :] = (hval * vval) * 100.0
            zcopies[mi].wait()
            sc = pltpu.make_async_copy(
                valbuf.at[slot], out_ref.at[idxbuf.at[slot]], ssem)
            sc.start()
            scopies.append(sc)
        for sc in scopies[-2:]:
            sc.wait()

    mesh = plsc.VectorSubcoreMesh(core_axis_name="c", subcore_axis_name="s")
    return functools.partial(
        pl.kernel, body, mesh=mesh,
        compiler_params=pltpu.CompilerParams(needs_layout_passes=False),
        out_type=jax.ShapeDtypeStruct((BP * mapsz,), jnp.float32),
        scratch_types=[
            pltpu.VMEM((maps_per_w * 4 * K,), jnp.float32),
            pltpu.VMEM((mapsz,), jnp.float32),
            pltpu.VMEM((2, _L), jnp.int32),
            pltpu.VMEM((2, _L), jnp.float32),
            pltpu.SemaphoreType.DMA,
            pltpu.SemaphoreType.DMA,
        ])()


def kernel(x_cat):
    B, A, K = x_cat.shape
    P = A // 2
    BP = B * P
    f32 = jnp.float32

    t = 0
    beta = 0.1 + 0.1 * np.sin(t / 1000)
    nkey = jax.random.key(42)
    noise = jnp.stack(
        [beta * jax.random.gumbel(jax.random.fold_in(nkey, j), (B, K), dtype=f32)
         for j in range(A)], axis=1)

    h = x_cat[:, 0::2, :]
    v = x_cat[:, 1::2, :]
    log_h = jnp.log(h) + noise[:, 0::2, :]
    log_v = jnp.log(v) + noise[:, 1::2, :]

    packed = jnp.stack(
        [h.reshape(BP, K), v.reshape(BP, K),
         log_h.reshape(BP, K), log_v.reshape(BP, K)], axis=1).reshape(-1)
    zeros = jnp.zeros((K * K,), f32)
    sampled = _make_sc_sampled(BP, K, BP // 32)(packed, zeros)

    hc = h[..., None]
    vr = v[:, :, None, :]
    places = pl.pallas_call(
        _places_body,
        grid=(B,),
        in_specs=[pl.BlockSpec((1, P, K, 1), lambda b: (b, 0, 0, 0)),
                  pl.BlockSpec((1, P, 1, K), lambda b: (b, 0, 0, 0))],
        out_specs=pl.BlockSpec((1, P, K, K), lambda b: (b, 0, 0, 0)),
        out_shape=jax.ShapeDtypeStruct((B, P, K, K), f32),
        compiler_params=pltpu.CompilerParams(
            dimension_semantics=("parallel",)),
    )(hc, vr)
    return (places, sampled.reshape(B, P, K, K))


# speedup vs baseline: 2.0679x; 2.0650x over previous
import functools

import numpy as np
import jax
import jax.numpy as jnp
from jax import lax
from jax.experimental import pallas as pl
from jax.experimental.pallas import tpu as pltpu
from jax.experimental.pallas import tpu_sc as plsc

_L = 16
_BIG = 1 << 22
_NOISE_CACHE = {}


def _gumbel_noise(B, A, K):
    if (B, A, K) not in _NOISE_CACHE:
        t = 0
        beta = 0.1 + 0.1 * np.sin(t / 1000)
        with jax.ensure_compile_time_eval():
            nkey = jax.random.key(42)
            _NOISE_CACHE[(B, A, K)] = np.stack(
                [np.asarray(beta * jax.random.gumbel(
                    jax.random.fold_in(nkey, j), (B, K), dtype=jnp.float32))
                 for j in range(A)], axis=1)
    return _NOISE_CACHE[(B, A, K)]


def _places_body(hc_ref, vr_ref, places_ref):
    places_ref[0] = hc_ref[0] * vr_ref[0]


def _make_sc_sampled(B, A, K, n_workers):
    P = A // 2
    BP = B * P
    maps_per_w = BP // n_workers
    batches_per_w = B // n_workers
    nchunks = K // _L

    def body(x_ref, lp_ref, zeros_ref, out_ref, xbuf, lpbuf, zbuf, rbuf,
             zsem, ssem):
        wid = lax.axis_index("c") * 16 + lax.axis_index("s")
        base_map = wid * maps_per_w
        pltpu.sync_copy(zeros_ref, zbuf)
        pltpu.sync_copy(x_ref.at[pl.ds(wid * batches_per_w, batches_per_w)],
                        xbuf)
        pltpu.sync_copy(lp_ref.at[pl.ds(wid * batches_per_w, batches_per_w)],
                        lpbuf)
        zcopies = []
        for mi in range(maps_per_w):
            c = pltpu.make_async_copy(zbuf, out_ref.at[base_map + mi], zsem)
            c.start()
            zcopies.append(c)

        lane = jax.lax.iota(jnp.int32, _L)
        zvec = jnp.zeros((_L,), jnp.float32)

        def winners(bl, row):
            mvec = jnp.full((_L,), -jnp.inf, jnp.float32)
            for c in range(nchunks):
                mvec = jnp.maximum(mvec, lpbuf[bl, row, pl.ds(c * _L, _L)])
            m = jnp.max(mvec)
            c1 = jnp.full((_L,), _BIG, jnp.int32)
            for c in range(nchunks):
                x = lpbuf[bl, row, pl.ds(c * _L, _L)]
                idx = lane + (c * _L)
                c1 = jnp.minimum(c1, jnp.where(x == m, idx, _BIG))
            i1 = jnp.min(c1)
            c2 = jnp.full((_L,), _BIG, jnp.int32)
            for c in range(nchunks):
                x = lpbuf[bl, row, pl.ds(c * _L, _L)]
                idx = lane + (c * _L)
                c2 = jnp.minimum(
                    c2, jnp.where((x == m) & (idx != i1), idx, _BIG))
            i2 = jnp.min(c2)
            return i1, i2

        scopies = [[], []]
        for mi in range(maps_per_w):
            bl, p = mi // P, mi % P
            i1, i2 = winners(bl, 2 * p)
            j1, j2 = winners(bl, 2 * p + 1)
            i2e = jnp.where(i2 < _BIG, i2, i1)
            j2e = jnp.where(j2 < _BIG, j2, j1)
            ri = jnp.where(((lane >> 1) & 1) == 1, i2e, i1)
            cj = jnp.where((lane & 1) == 1, j2e, j1)
            blv = jnp.full((_L,), bl, jnp.int32)
            hval = plsc.load_gather(xbuf, [blv, jnp.full((_L,), 2 * p,
                                                         jnp.int32), ri])
            vval = plsc.load_gather(xbuf, [blv, jnp.full((_L,), 2 * p + 1,
                                                         jnp.int32), cj])
            slot = mi % 2
            for sc in scopies[slot]:
                sc.wait()
            scopies[slot] = []
            for c in range(nchunks):
                rbuf[slot, 0, pl.ds(c * _L, _L)] = zvec
                rbuf[slot, 1, pl.ds(c * _L, _L)] = zvec
            plsc.store_scatter(
                rbuf, [jnp.full((_L,), slot, jnp.int32), (lane >> 1) & 1, cj],
                (hval * vval) * 100.0)
            zcopies[mi].wait()
            m = base_map + mi
            for rr, rowidx in ((0, i1), (1, i2e)):
                for cb in range(K // 128):
                    sc = pltpu.make_async_copy(
                        rbuf.at[slot, rr, pl.ds(cb * 128, 128)],
                        out_ref.at[m, rowidx, pl.ds(cb * 128, 128)], ssem)
                    sc.start()
                    scopies[slot].append(sc)
        for slot in (0, 1):
            for sc in scopies[slot]:
                sc.wait()

    mesh = plsc.VectorSubcoreMesh(core_axis_name="c", subcore_axis_name="s")
    return functools.partial(
        pl.kernel, body, mesh=mesh,
        compiler_params=pltpu.CompilerParams(needs_layout_passes=False),
        out_type=jax.ShapeDtypeStruct((BP, K, K), jnp.float32),
        scratch_types=[
            pltpu.VMEM((batches_per_w, A, K), jnp.float32),
            pltpu.VMEM((batches_per_w, A, K), jnp.float32),
            pltpu.VMEM((K, K), jnp.float32),
            pltpu.VMEM((2, 2, K), jnp.float32),
            pltpu.SemaphoreType.DMA,
            pltpu.SemaphoreType.DMA,
        ])()


def kernel(x_cat):
    B, A, K = x_cat.shape
    P = A // 2
    f32 = jnp.float32

    try:
        noise = jnp.asarray(_gumbel_noise(B, A, K))
    except Exception:
        t = 0
        beta = 0.1 + 0.1 * np.sin(t / 1000)
        nkey = jax.random.key(42)
        noise = jnp.stack(
            [beta * jax.random.gumbel(jax.random.fold_in(nkey, j), (B, K),
                                      dtype=f32)
             for j in range(A)], axis=1)
    lp = jnp.log(x_cat) + noise

    zeros = jnp.zeros((K, K), f32)
    sampled = _make_sc_sampled(B, A, K, 32)(x_cat, lp, zeros)

    hc = x_cat[:, 0::2, :, None]
    vr = x_cat[:, 1::2, None, :]
    places = pl.pallas_call(
        _places_body,
        grid=(B,),
        in_specs=[pl.BlockSpec((1, P, K, 1), lambda b: (b, 0, 0, 0)),
                  pl.BlockSpec((1, P, 1, K), lambda b: (b, 0, 0, 0))],
        out_specs=pl.BlockSpec((1, P, K, K), lambda b: (b, 0, 0, 0)),
        out_shape=jax.ShapeDtypeStruct((B, P, K, K), f32),
        compiler_params=pltpu.CompilerParams(
            dimension_semantics=("parallel",)),
    )(hc, vr)
    return (places, sampled.reshape(B, P, K, K))


# speedup vs baseline: 2.8498x; 1.3781x over previous
import functools

import numpy as np
import jax
import jax.numpy as jnp
from jax import lax
from jax.experimental import pallas as pl
from jax.experimental.pallas import tpu as pltpu
from jax.experimental.pallas import tpu_sc as plsc

_L = 16
_BIG = 1 << 22
_NOISE_CACHE = {}


def _gumbel_noise(B, A, K):
    if (B, A, K) not in _NOISE_CACHE:
        t = 0
        beta = 0.1 + 0.1 * np.sin(t / 1000)
        with jax.ensure_compile_time_eval():
            nkey = jax.random.key(42)
            _NOISE_CACHE[(B, A, K)] = np.stack(
                [np.asarray(beta * jax.random.gumbel(
                    jax.random.fold_in(nkey, j), (B, K), dtype=jnp.float32))
                 for j in range(A)], axis=1)
    return _NOISE_CACHE[(B, A, K)]


def _make_places_body(P):
    def body(x_ref, places_ref):
        for p in range(P):
            h = x_ref[0, pl.ds(2 * p, 1), :]
            v = x_ref[0, pl.ds(2 * p + 1, 1), :]
            places_ref[0, p] = lax.dot_general(
                h, v, (((0,), (0,)), ((), ())),
                preferred_element_type=jnp.float32)
    return body


def _make_sc_sampled(B, A, K, n_workers):
    P = A // 2
    BP = B * P
    maps_per_w = BP // n_workers
    batches_per_w = B // n_workers
    nchunks = K // _L

    def body(x_ref, lp_ref, zeros_ref, out_ref, xbuf, lpbuf, zbuf, rbuf,
             zsem, ssem):
        wid = lax.axis_index("c") * 16 + lax.axis_index("s")
        base_map = wid * maps_per_w
        pltpu.sync_copy(zeros_ref, zbuf)
        pltpu.sync_copy(x_ref.at[pl.ds(wid * batches_per_w, batches_per_w)],
                        xbuf)
        pltpu.sync_copy(lp_ref.at[pl.ds(wid * batches_per_w, batches_per_w)],
                        lpbuf)
        zcopies = []
        for mi in range(maps_per_w):
            c = pltpu.make_async_copy(zbuf, out_ref.at[base_map + mi], zsem)
            c.start()
            zcopies.append(c)

        lane = jax.lax.iota(jnp.int32, _L)
        zvec = jnp.zeros((_L,), jnp.float32)

        def winners(bl, row):
            mvec = jnp.full((_L,), -jnp.inf, jnp.float32)
            for c in range(nchunks):
                mvec = jnp.maximum(mvec, lpbuf[bl, row, pl.ds(c * _L, _L)])
            m = jnp.max(mvec)
            c1 = jnp.full((_L,), _BIG, jnp.int32)
            for c in range(nchunks):
                x = lpbuf[bl, row, pl.ds(c * _L, _L)]
                idx = lane + (c * _L)
                c1 = jnp.minimum(c1, jnp.where(x == m, idx, _BIG))
            i1 = jnp.min(c1)
            c2 = jnp.full((_L,), _BIG, jnp.int32)
            for c in range(nchunks):
                x = lpbuf[bl, row, pl.ds(c * _L, _L)]
                idx = lane + (c * _L)
                c2 = jnp.minimum(
                    c2, jnp.where((x == m) & (idx != i1), idx, _BIG))
            i2 = jnp.min(c2)
            return i1, i2

        scopies = [[], []]
        for mi in range(maps_per_w):
            bl, p = mi // P, mi % P
            i1, i2 = winners(bl, 2 * p)
            j1, j2 = winners(bl, 2 * p + 1)
            i2e = jnp.where(i2 < _BIG, i2, i1)
            j2e = jnp.where(j2 < _BIG, j2, j1)
            ri = jnp.where(((lane >> 1) & 1) == 1, i2e, i1)
            cj = jnp.where((lane & 1) == 1, j2e, j1)
            blv = jnp.full((_L,), bl, jnp.int32)
            hval = plsc.load_gather(xbuf, [blv, jnp.full((_L,), 2 * p,
                                                         jnp.int32), ri])
            vval = plsc.load_gather(xbuf, [blv, jnp.full((_L,), 2 * p + 1,
                                                         jnp.int32), cj])
            slot = mi % 2
            for sc in scopies[slot]:
                sc.wait()
            scopies[slot] = []
            for c in range(nchunks):
                rbuf[slot, 0, pl.ds(c * _L, _L)] = zvec
                rbuf[slot, 1, pl.ds(c * _L, _L)] = zvec
            plsc.store_scatter(
                rbuf, [jnp.full((_L,), slot, jnp.int32), (lane >> 1) & 1, cj],
                (hval * vval) * 100.0)
            zcopies[mi].wait()
            m = base_map + mi
            for rr, rowidx in ((0, i1), (1, i2e)):
                for cb in range(K // 128):
                    sc = pltpu.make_async_copy(
                        rbuf.at[slot, rr, pl.ds(cb * 128, 128)],
                        out_ref.at[m, rowidx, pl.ds(cb * 128, 128)], ssem)
                    sc.start()
                    scopies[slot].append(sc)
        for slot in (0, 1):
            for sc in scopies[slot]:
                sc.wait()

    mesh = plsc.VectorSubcoreMesh(core_axis_name="c", subcore_axis_name="s")
    return functools.partial(
        pl.kernel, body, mesh=mesh,
        compiler_params=pltpu.CompilerParams(needs_layout_passes=False),
        out_type=jax.ShapeDtypeStruct((BP, K, K), jnp.float32),
        scratch_types=[
            pltpu.VMEM((batches_per_w, A, K), jnp.float32),
            pltpu.VMEM((batches_per_w, A, K), jnp.float32),
            pltpu.VMEM((K, K), jnp.float32),
            pltpu.VMEM((2, 2, K), jnp.float32),
            pltpu.SemaphoreType.DMA,
            pltpu.SemaphoreType.DMA,
        ])()


def kernel(x_cat):
    B, A, K = x_cat.shape
    P = A // 2
    f32 = jnp.float32

    try:
        noise = jnp.asarray(_gumbel_noise(B, A, K))
    except Exception:
        t = 0
        beta = 0.1 + 0.1 * np.sin(t / 1000)
        nkey = jax.random.key(42)
        noise = jnp.stack(
            [beta * jax.random.gumbel(jax.random.fold_in(nkey, j), (B, K),
                                      dtype=f32)
             for j in range(A)], axis=1)
    lp = jnp.log(x_cat) + noise

    zeros = jnp.zeros((K, K), f32)
    sampled = _make_sc_sampled(B, A, K, 32)(x_cat, lp, zeros)

    places = pl.pallas_call(
        _make_places_body(P),
        grid=(B,),
        in_specs=[pl.BlockSpec((1, A, K), lambda b: (b, 0, 0))],
        out_specs=pl.BlockSpec((1, P, K, K), lambda b: (b, 0, 0, 0)),
        out_shape=jax.ShapeDtypeStruct((B, P, K, K), f32),
        compiler_params=pltpu.CompilerParams(
            dimension_semantics=("parallel",)),
    )(x_cat)
    return (places, sampled.reshape(B, P, K, K))
